# baseline (speedup 1.0000x reference)
import functools

import jax
import jax.numpy as jnp
from jax import lax
from jax.experimental import pallas as pl
from jax.experimental.pallas import tpu as pltpu
from jax.experimental.pallas import tpu_sc as plsc


def _gather_rows_sc(emb_table, center_words):
    B = center_words.shape[0]
    D = emb_table.shape[1]
    info = plsc.get_sparse_core_info()
    nc, ns = info.num_cores, info.num_subcores
    nw = nc * ns
    b_per_w = B // nw
    mesh = plsc.VectorSubcoreMesh(core_axis_name="c", subcore_axis_name="s")

    @functools.partial(
        pl.kernel,
        mesh=mesh,
        out_type=jax.ShapeDtypeStruct((B, D), jnp.float32),
        scratch_types=[
            pltpu.VMEM((b_per_w,), jnp.int32),
            pltpu.VMEM((b_per_w, D), jnp.float32),
            pltpu.SemaphoreType.DMA,
        ],
        compiler_params=pltpu.CompilerParams(use_tc_tiling_on_sc=False),
    )
    def gather_kernel(idx_hbm, table_hbm, out_hbm, idx_v, rows_v, sem):
        wid = lax.axis_index("s") * nc + lax.axis_index("c")
        base = wid * b_per_w
        pltpu.sync_copy(idx_hbm.at[pl.ds(base, b_per_w)], idx_v)
        pltpu.async_copy(table_hbm.at[idx_v], rows_v, sem).wait()
        pltpu.sync_copy(rows_v, out_hbm.at[pl.ds(base, b_per_w)])

    return gather_kernel(center_words.astype(jnp.int32), emb_table)


def _decode_tc(e, dec_W, dec_b, v_tile=2048):
    B, D = e.shape
    V = dec_W.shape[0]
    grid = pl.cdiv(V, v_tile)
    b2 = dec_b.reshape(1, V)

    def body(e_ref, w_ref, b_ref, out_ref):
        out_ref[...] = (
            lax.dot_general(
                e_ref[...],
                w_ref[...],
                dimension_numbers=(((1,), (1,)), ((), ())),
                preferred_element_type=jnp.float32,
            )
            + b_ref[...]
        )

    return pl.pallas_call(
        body,
        grid=(grid,),
        in_specs=[
            pl.BlockSpec((B, D), lambda i: (0, 0)),
            pl.BlockSpec((v_tile, D), lambda i: (i, 0)),
            pl.BlockSpec((1, v_tile), lambda i: (0, i)),
        ],
        out_specs=pl.BlockSpec((B, v_tile), lambda i: (0, i)),
        out_shape=jax.ShapeDtypeStruct((B, V), jnp.float32),
    )(e, dec_W, b2)


def kernel(center_words, emb_table, dec_W, dec_b):
    e = _gather_rows_sc(emb_table, center_words)
    return _decode_tc(e, dec_W, dec_b)


# speedup vs baseline: 1.0001x; 1.0001x over previous
import functools

import jax
import jax.numpy as jnp
from jax import lax
from jax.experimental import pallas as pl
from jax.experimental.pallas import tpu as pltpu
from jax.experimental.pallas import tpu_sc as plsc


def _gather_rows_sc(emb_table, center_words):
    B = center_words.shape[0]
    D = emb_table.shape[1]
    info = plsc.get_sparse_core_info()
    nc, ns = info.num_cores, info.num_subcores
    nw = nc * ns
    b_per_w = B // nw
    mesh = plsc.VectorSubcoreMesh(core_axis_name="c", subcore_axis_name="s")

    @functools.partial(
        pl.kernel,
        mesh=mesh,
        out_type=jax.ShapeDtypeStruct((B, D), jnp.float32),
        scratch_types=[
            pltpu.VMEM((b_per_w,), jnp.int32),
            pltpu.VMEM((b_per_w, D), jnp.float32),
            pltpu.SemaphoreType.DMA,
        ],
        compiler_params=pltpu.CompilerParams(use_tc_tiling_on_sc=False),
    )
    def gather_kernel(idx_hbm, table_hbm, out_hbm, idx_v, rows_v, sem):
        wid = lax.axis_index("s") * nc + lax.axis_index("c")
        base = wid * b_per_w
        pltpu.sync_copy(idx_hbm.at[pl.ds(base, b_per_w)], idx_v)
        pltpu.async_copy(table_hbm.at[idx_v], rows_v, sem).wait()
        pltpu.sync_copy(rows_v, out_hbm.at[pl.ds(base, b_per_w)])

    return gather_kernel(center_words.astype(jnp.int32), emb_table)


def _decode_tc(e, dec_W, dec_b, v_tile=2048, nbuf=4, stripes=4):
    B, D = e.shape
    V = dec_W.shape[0]
    grid = pl.cdiv(V, v_tile)
    v_tail = ((V - (grid - 1) * v_tile) + 127) // 128 * 128
    rs = B // stripes
    b2 = dec_b.reshape(1, V)
    last_slot = (grid - 1) % nbuf

    def body(e_ref, w_ref, b_ref, out_hbm, acc, sems):
        i = pl.program_id(0)
        slot = lax.rem(i, nbuf)

        @pl.when(i >= nbuf)
        def _():
            for s in range(stripes):
                pltpu.make_async_copy(
                    acc.at[slot, pl.ds(s * rs, rs), :],
                    out_hbm.at[pl.ds(s * rs, rs), pl.ds((i - nbuf) * v_tile, v_tile)],
                    sems.at[slot, s],
                ).wait()

        acc[slot] = (
            lax.dot_general(
                e_ref[...],
                w_ref[...],
                dimension_numbers=(((1,), (1,)), ((), ())),
                preferred_element_type=jnp.float32,
            )
            + b_ref[...]
        )

        @pl.when(i < grid - 1)
        def _():
            for s in range(stripes):
                pltpu.make_async_copy(
                    acc.at[slot, pl.ds(s * rs, rs), :],
                    out_hbm.at[pl.ds(s * rs, rs), pl.ds(i * v_tile, v_tile)],
                    sems.at[slot, s],
                ).start()

        @pl.when(i == grid - 1)
        def _():
            for s in range(stripes):
                pltpu.make_async_copy(
                    acc.at[slot, pl.ds(s * rs, rs), pl.ds(0, v_tail)],
                    out_hbm.at[pl.ds(s * rs, rs), pl.ds(i * v_tile, v_tail)],
                    sems.at[slot, s],
                ).start()
            for j in range(nbuf):
                w = v_tail if j == last_slot else v_tile
                for s in range(stripes):
                    pltpu.make_async_copy(
                        acc.at[j, pl.ds(s * rs, rs), pl.ds(0, w)],
                        out_hbm.at[pl.ds(s * rs, rs), pl.ds(0, w)],
                        sems.at[j, s],
                    ).wait()

    return pl.pallas_call(
        body,
        grid=(grid,),
        in_specs=[
            pl.BlockSpec((B, D), lambda i: (0, 0)),
            pl.BlockSpec((v_tile, D), lambda i: (i, 0)),
            pl.BlockSpec((1, v_tile), lambda i: (0, i)),
        ],
        out_specs=pl.BlockSpec(memory_space=pl.ANY),
        out_shape=jax.ShapeDtypeStruct((B, V), jnp.float32),
        scratch_shapes=[
            pltpu.VMEM((nbuf, B, v_tile), jnp.float32),
            pltpu.SemaphoreType.DMA((nbuf, stripes)),
        ],
    )(e, dec_W, b2)


def kernel(center_words, emb_table, dec_W, dec_b):
    e = _gather_rows_sc(emb_table, center_words)
    return _decode_tc(e, dec_W, dec_b)


# speedup vs baseline: 2.8259x; 2.8257x over previous
import functools

import jax
import jax.numpy as jnp
from jax import lax
from jax.experimental import pallas as pl
from jax.experimental.pallas import tpu as pltpu
from jax.experimental.pallas import tpu_sc as plsc


def _gather_rows_sc(emb_table, center_words):
    B = center_words.shape[0]
    D = emb_table.shape[1]
    info = plsc.get_sparse_core_info()
    nc, ns = info.num_cores, info.num_subcores
    nw = nc * ns
    b_per_w = B // nw
    mesh = plsc.VectorSubcoreMesh(core_axis_name="c", subcore_axis_name="s")

    @functools.partial(
        pl.kernel,
        mesh=mesh,
        out_type=jax.ShapeDtypeStruct((B, D), jnp.float32),
        scratch_types=[
            pltpu.VMEM((b_per_w,), jnp.int32),
            pltpu.VMEM((b_per_w, D), jnp.float32),
            pltpu.SemaphoreType.DMA,
        ],
        compiler_params=pltpu.CompilerParams(use_tc_tiling_on_sc=False),
    )
    def gather_kernel(idx_hbm, table_hbm, out_hbm, idx_v, rows_v, sem):
        wid = lax.axis_index("s") * nc + lax.axis_index("c")
        base = wid * b_per_w
        pltpu.sync_copy(idx_hbm.at[pl.ds(base, b_per_w)], idx_v)
        pltpu.async_copy(table_hbm.at[idx_v], rows_v, sem).wait()
        pltpu.sync_copy(rows_v, out_hbm.at[pl.ds(base, b_per_w)])

    return gather_kernel(center_words.astype(jnp.int32), emb_table)


def _decode_tc(e, dec_Wt, dec_b, v_tile=2048):
    B, D = e.shape
    V = dec_Wt.shape[1]
    grid = pl.cdiv(V, v_tile)
    b2 = dec_b.reshape(1, V)

    def body(e_ref, wt_ref, b_ref, out_ref):
        prod = lax.dot_general(
            wt_ref[...],
            e_ref[...],
            dimension_numbers=(((0,), (1,)), ((), ())),
            preferred_element_type=jnp.float32,
        )
        ones = jnp.ones((1, B), dtype=jnp.float32)
        bias = lax.dot_general(
            b_ref[...],
            ones,
            dimension_numbers=(((0,), (0,)), ((), ())),
            preferred_element_type=jnp.float32,
        )
        out_ref[...] = prod + bias

    return pl.pallas_call(
        body,
        grid=(grid,),
        in_specs=[
            pl.BlockSpec((B, D), lambda i: (0, 0)),
            pl.BlockSpec((D, v_tile), lambda i: (0, i)),
            pl.BlockSpec((1, v_tile), lambda i: (0, i)),
        ],
        out_specs=pl.BlockSpec((v_tile, B), lambda i: (i, 0)),
        out_shape=jax.ShapeDtypeStruct((V, B), jnp.float32),
    )(e, dec_Wt, b2)


def kernel(center_words, emb_table, dec_W, dec_b):
    e = _gather_rows_sc(emb_table, center_words)
    out_t = _decode_tc(e, dec_W.T, dec_b)
    return out_t.T


# speedup vs baseline: 2.9628x; 1.0485x over previous
import functools

import jax
import jax.numpy as jnp
from jax import lax
from jax.experimental import pallas as pl
from jax.experimental.pallas import tpu as pltpu
from jax.experimental.pallas import tpu_sc as plsc


def _gather_rows_sc(emb_table_128, center_words, d_valid):
    B = center_words.shape[0]
    D = emb_table_128.shape[1]
    info = plsc.get_sparse_core_info()
    nc, ns = info.num_cores, info.num_subcores
    nw = nc * ns
    b_per_w = B // nw
    mesh = plsc.VectorSubcoreMesh(core_axis_name="c", subcore_axis_name="s")

    del d_valid

    @functools.partial(
        pl.kernel,
        mesh=mesh,
        out_type=jax.ShapeDtypeStruct((B, D), jnp.float32),
        scratch_types=[
            pltpu.VMEM((b_per_w,), jnp.int32),
            pltpu.VMEM((b_per_w, D), jnp.float32),
            pltpu.SemaphoreType.DMA,
        ],
    )
    def gather_kernel(idx_hbm, table_hbm, out_hbm, idx_v, rows_v, sem):
        wid = lax.axis_index("s") * nc + lax.axis_index("c")
        base = wid * b_per_w
        pltpu.sync_copy(idx_hbm.at[pl.ds(base, b_per_w)], idx_v)
        pltpu.async_copy(table_hbm.at[idx_v], rows_v, sem).wait()
        pltpu.sync_copy(rows_v, out_hbm.at[pl.ds(base, b_per_w)])

    return gather_kernel(center_words.astype(jnp.int32), emb_table_128)


def _decode_tc(e, dec_Wt, dec_b, v_tile=2048):
    B = e.shape[0]
    D, V = dec_Wt.shape
    grid = pl.cdiv(V, v_tile)
    b2 = dec_b.reshape(1, V)

    def body(e_ref, wt_ref, b_ref, out_ref):
        prod = lax.dot_general(
            wt_ref[...],
            e_ref[...][:, :D],
            dimension_numbers=(((0,), (1,)), ((), ())),
            preferred_element_type=jnp.float32,
        )
        ones = jnp.ones((1, B), dtype=jnp.float32)
        bias = lax.dot_general(
            b_ref[...],
            ones,
            dimension_numbers=(((0,), (0,)), ((), ())),
            preferred_element_type=jnp.float32,
        )
        out_ref[...] = prod + bias

    return pl.pallas_call(
        body,
        grid=(grid,),
        in_specs=[
            pl.BlockSpec((B, e.shape[1]), lambda i: (0, 0)),
            pl.BlockSpec((D, v_tile), lambda i: (0, i)),
            pl.BlockSpec((1, v_tile), lambda i: (0, i)),
        ],
        out_specs=pl.BlockSpec((v_tile, B), lambda i: (i, 0)),
        out_shape=jax.ShapeDtypeStruct((V, B), jnp.float32),
    )(e, dec_Wt, b2)


def kernel(center_words, emb_table, dec_W, dec_b):
    d = emb_table.shape[1]
    emb128 = jnp.pad(emb_table, ((0, 0), (0, 128 - d)))
    e = _gather_rows_sc(emb128, center_words, d)
    out_t = _decode_tc(e, dec_W.T, dec_b)
    return out_t.T
